# baseline (speedup 1.0000x reference)
import functools

import jax
import jax.numpy as jnp
from jax import lax
from jax.experimental import pallas as pl
from jax.experimental.pallas import tpu as pltpu
from jax.experimental.pallas import tpu_sc as plsc

D_MODEL = 1024
SCALE = 32.0
LANES = 16

NC = 2
NS = 16
NW = NC * NS

B_TOTAL = 4 * 4096
B_PER_W = B_TOTAL // NW
CHUNK = 32
HALF = CHUNK // 2
NCH = B_PER_W // CHUNK

_mesh = plsc.VectorSubcoreMesh(core_axis_name="c", subcore_axis_name="s")


def _scale_half(src, h, dst):
  def row(r, _):
    def col(i, _):
      sl = pl.ds(i * LANES, LANES)
      dst[r, sl] = src[h * HALF + r, sl] * SCALE
      return 0
    return lax.fori_loop(0, D_MODEL // LANES, col, 0, unroll=4)
  lax.fori_loop(0, HALF, row, 0)


@functools.partial(
    pl.kernel,
    out_type=jax.ShapeDtypeStruct((B_TOTAL, D_MODEL), jnp.float32),
    mesh=_mesh,
    scratch_types=(
        [pltpu.VMEM((NCH, CHUNK), jnp.int32)]
        + [pltpu.VMEM((CHUNK, D_MODEL), jnp.float32)] * 2
        + [pltpu.VMEM((HALF, D_MODEL), jnp.float32)] * 2
        + [pltpu.SemaphoreType.DMA] * 4
    ),
)
def _emb_lookup(x_hbm, table_hbm, out_hbm, idx_v, i0, i1, o0, o1,
                g0, g1, s0, s1):
  ins = (i0, i1)
  outs = (o0, o1)
  gsems = (g0, g1)
  ssems = (s0, s1)

  wid = lax.axis_index("s") * NC + lax.axis_index("c")
  base = wid * B_PER_W

  pltpu.sync_copy(x_hbm.at[wid], idx_v)

  for b in range(2):
    pltpu.async_copy(table_hbm.at[idx_v.at[b]], ins[b], gsems[b])

  pltpu.make_async_copy(table_hbm.at[idx_v.at[0]], ins[0], gsems[0]).wait()
  for h in range(2):
    _scale_half(ins[0], h, outs[h])
    pltpu.async_copy(
        outs[h], out_hbm.at[pl.ds(base + h * HALF, HALF)], ssems[h])
  pltpu.async_copy(table_hbm.at[idx_v.at[2]], ins[0], gsems[0])

  pltpu.make_async_copy(table_hbm.at[idx_v.at[1]], ins[1], gsems[1]).wait()
  for h in range(2):
    pltpu.make_async_copy(
        outs[h], out_hbm.at[pl.ds(base, HALF)], ssems[h]).wait()
    _scale_half(ins[1], h, outs[h])
    pltpu.async_copy(
        outs[h], out_hbm.at[pl.ds(base + CHUNK + h * HALF, HALF)], ssems[h])
  pltpu.async_copy(table_hbm.at[idx_v.at[3]], ins[1], gsems[1])

  def group(g, _):
    for b in range(2):
      j = g * 2 + 2 + b
      bi = b
      pltpu.make_async_copy(
          table_hbm.at[idx_v.at[j]], ins[bi], gsems[bi]).wait()
      for h in range(2):
        pltpu.make_async_copy(
            outs[h], out_hbm.at[pl.ds(base, HALF)], ssems[h]).wait()
        _scale_half(ins[bi], h, outs[h])
        pltpu.async_copy(
            outs[h],
            out_hbm.at[pl.ds(base + j * CHUNK + h * HALF, HALF)], ssems[h])
      nj = j + 2

      @pl.when(nj < NCH)
      def _():
        pltpu.async_copy(table_hbm.at[idx_v.at[nj]], ins[bi], gsems[bi])
    return 0

  lax.fori_loop(0, (NCH - 2) // 2, group, 0)

  for h in range(2):
    pltpu.make_async_copy(
        outs[h], out_hbm.at[pl.ds(base, HALF)], ssems[h]).wait()


def kernel(x, table):
  xf = x.astype(jnp.int32).reshape(NW, NCH, CHUNK)
  out = _emb_lookup(xf, table)
  return out.reshape(x.shape + (D_MODEL,))
